# baseline (speedup 1.0000x reference)
import functools

import jax
import jax.numpy as jnp
import numpy as np
from jax import lax
from jax.experimental import pallas as pl
from jax.experimental.pallas import tpu as pltpu
from jax.experimental.pallas import tpu_sc as plsc

N = 100000
HALF = N // 2
K = 16
L = 16
NC = 2
NS = 16
CN = 128
CG = CN // L
CE = CN * K
NSLOT = -(-N // CN)
TAIL_NB = N - CN
ROUNDS = -(-(-(-NSLOT // NC)) // NS)
MAXA = 8

DT = np.float32(0.01)
EPS = np.float32(1e-14)
GROUND = np.float32(-2.0)
REBOUND = np.float32(0.1)
GRAV_Y = np.float32(-9.8)
LN10 = np.float32(2.302585092994046)


def _rsqrt(x):
    i = lax.bitcast_convert_type(x, jnp.int32)
    i = np.int32(0x5F3759DF) - lax.shift_right_logical(i, 1)
    y = lax.bitcast_convert_type(i, jnp.float32)
    for _ in range(3):
        y = y * (np.float32(1.5) - np.float32(0.5) * x * y * y)
    return y


_mesh = plsc.VectorSubcoreMesh(core_axis_name="c", subcore_axis_name="s")


@functools.partial(
    pl.kernel,
    out_type=jax.ShapeDtypeStruct((N * 6,), jnp.float32),
    mesh=_mesh,
    compiler_params=pltpu.CompilerParams(needs_layout_passes=False),
    scratch_types=[
        pltpu.VMEM((HALF,), jnp.float32),
        pltpu.VMEM((2 * CE,), jnp.int32),
        pltpu.VMEM((2 * CE,), jnp.float32),
        pltpu.VMEM((CE,), jnp.float32),
        pltpu.VMEM((CE,), jnp.float32),
        pltpu.VMEM((CE,), jnp.float32),
        pltpu.VMEM((CE,), jnp.float32),
        pltpu.VMEM((CE,), jnp.float32),
        pltpu.VMEM((CE,), jnp.float32),
        pltpu.VMEM((CN * 3,), jnp.float32),
        pltpu.VMEM((CN * 3,), jnp.float32),
        pltpu.VMEM((CE,), jnp.float32),
        pltpu.VMEM((CE,), jnp.float32),
        pltpu.VMEM((CN,), jnp.float32),
        pltpu.VMEM((CN * 6,), jnp.float32),
        pltpu.VMEM_SHARED((2 * NS * 6 * CE,), jnp.float32),
        [pltpu.SemaphoreType.DMA for _ in range(2)],
        [pltpu.SemaphoreType.DMA for _ in range(2)],
        pltpu.SemaphoreType.DMA,
        pltpu.SemaphoreType.DMA,
        pltpu.SemaphoreType.DMA,
    ],
)
def _sc_step(xs, ys, zs, xyzf, velf, olf, gkf, gm, knnf, out,
             col_v, idx_v, gout_v,
             rxl_v, rxh_v, ryl_v, ryh_v, rzl_v, rzh_v,
             own_v, vel_v, ol_v, gk_v, gm_v, out_v, G,
             ksem, gwsem, rbsem, lsem, osem):
    cid = lax.axis_index("c")
    sid = lax.axis_index("s")
    ii = lax.iota(jnp.int32, L)

    col = jnp.where(sid < 6, 0, jnp.where(sid < 12, 1, 2))
    tbase = jnp.where(sid < 6, 0, jnp.where(sid < 12, 6, 12))
    loc = sid - tbase
    tsz = jnp.where(sid < 12, 3, 2)
    half = jnp.where(loc >= tsz, 1, 0)
    tmem = loc - half * tsz
    hbase = half * HALF

    def node_base(t):
        nb = jnp.where(t == NSLOT - 1, TAIL_NB, t * CN)
        return pl.multiple_of(nb, 32)

    def region(buf, q, c, h):
        off = ((((buf * NS) + q) * 3 + c) * 2 + h) * CE
        return pl.multiple_of(off, CE)

    hoff = pl.multiple_of(half * HALF, 8)

    @pl.when(col == 0)
    def _():
        pltpu.sync_copy(xs.at[pl.ds(hoff, HALF)], col_v)

    @pl.when(col == 1)
    def _():
        pltpu.sync_copy(ys.at[pl.ds(hoff, HALF)], col_v)

    @pl.when(col == 2)
    def _():
        pltpu.sync_copy(zs.at[pl.ds(hoff, HALF)], col_v)

    def assign(r, a):
        q = tmem + a * tsz
        t = (r * NS + q) * NC + cid
        return q, t, jnp.logical_and(q < NS, t < NSLOT)

    def fire_knn(r, a):
        q, t, valid = assign(r, a)

        @pl.when(valid)
        def _():
            nb = node_base(t)
            pltpu.make_async_copy(
                knnf.at[pl.ds(nb * K, CE)],
                idx_v.at[pl.ds((a % 2) * CE, CE)], ksem[a % 2]).start()

    def gather_round(r):
        buf = r % 2
        fire_knn(r, 0)
        for a in range(MAXA):
            if a + 1 < MAXA:
                fire_knn(r, a + 1)
            q, t, valid = assign(r, a)
            p = a % 2

            @pl.when(valid)
            def _():
                pltpu.make_async_copy(
                    knnf.at[pl.ds(0, CE)],
                    idx_v.at[pl.ds(0, CE)], ksem[p]).wait()
                if a >= 2:
                    pltpu.make_async_copy(
                        gout_v.at[pl.ds(0, CE)],
                        G.at[pl.ds(0, CE)], gwsem[p]).wait()

                def gather_body(it, u):
                    base = it * (4 * L)
                    for w in range(4):
                        pos = ii + (base + w * L)
                        kv = plsc.load_gather(idx_v, [pos + p * CE])
                        kvb = kv - hbase
                        m = jnp.logical_and(kvb >= 0, kvb < HALF)
                        cl = jnp.minimum(jnp.maximum(kvb, 0), HALF - 1)
                        vals = plsc.load_gather(col_v, [cl])
                        vals = jnp.where(m, vals, np.float32(0.0))
                        plsc.store_scatter(gout_v, [pos + p * CE], vals)
                    return u

                lax.fori_loop(0, CE // (4 * L), gather_body, 0)
                pltpu.make_async_copy(
                    gout_v.at[pl.ds(p * CE, CE)],
                    G.at[pl.ds(region(buf, q, col, half), CE)],
                    gwsem[p]).start()

        for a in range(MAXA):
            q, t, valid = assign(r, a)
            _, _, valid2 = assign(r, a + 2)

            @pl.when(jnp.logical_and(valid, jnp.logical_not(valid2)))
            def _():
                pltpu.make_async_copy(
                    gout_v.at[pl.ds(0, CE)],
                    G.at[pl.ds(0, CE)], gwsem[a % 2]).wait()

    def fire_compute_loads(r):
        buf = r % 2
        t = (r * NS + sid) * NC + cid
        nb = node_base(t)
        for i, dst in enumerate((rxl_v, rxh_v, ryl_v, ryh_v, rzl_v, rzh_v)):
            pltpu.make_async_copy(
                G.at[pl.ds(region(buf, sid, i // 2, i % 2), CE)],
                dst, rbsem).start()
        pltpu.make_async_copy(
            xyzf.at[pl.ds(nb * 3, CN * 3)], own_v, lsem).start()
        pltpu.make_async_copy(
            velf.at[pl.ds(nb * 3, CN * 3)], vel_v, lsem).start()
        pltpu.make_async_copy(olf.at[pl.ds(nb * K, CE)], ol_v, lsem).start()
        pltpu.make_async_copy(gkf.at[pl.ds(nb * K, CE)], gk_v, lsem).start()
        pltpu.make_async_copy(gm.at[pl.ds(nb, CN)], gm_v, lsem).start()

    def compute_round(r):
        t = (r * NS + sid) * NC + cid
        nb = node_base(t)
        for _ in range(6):
            pltpu.make_async_copy(
                G.at[pl.ds(0, CE)], rxl_v, rbsem).wait()
        for nw, dref in ((CN * 3, own_v), (CN * 3, vel_v), (CE, ol_v),
                         (CE, gk_v), (CN, gm_v)):
            pltpu.make_async_copy(gm.at[pl.ds(0, nw)], dref, lsem).wait()

        @pl.when(r >= 1)
        def _():
            pltpu.make_async_copy(
                out_v, out.at[pl.ds(nb * 6, CN * 6)], osem).wait()

        def group_body(gr, u):
            rr = gr * L + ii
            r3 = rr * 3
            ox = plsc.load_gather(own_v, [r3])
            oy = plsc.load_gather(own_v, [r3 + 1])
            oz = plsc.load_gather(own_v, [r3 + 2])
            vx = plsc.load_gather(vel_v, [r3])
            vy = plsc.load_gather(vel_v, [r3 + 1])
            vz = plsc.load_gather(vel_v, [r3 + 2])
            mlg = plsc.load_gather(gm_v, [rr])

            ax = jnp.zeros((L,), jnp.float32)
            ay = jnp.zeros((L,), jnp.float32)
            az = jnp.zeros((L,), jnp.float32)
            rk = rr * K
            for j in range(K):
                flat = rk + j
                nx = (plsc.load_gather(rxl_v, [flat])
                      + plsc.load_gather(rxh_v, [flat]))
                ny = (plsc.load_gather(ryl_v, [flat])
                      + plsc.load_gather(ryh_v, [flat]))
                nz = (plsc.load_gather(rzl_v, [flat])
                      + plsc.load_gather(rzh_v, [flat]))
                olj = plsc.load_gather(ol_v, [flat])
                kj = plsc.load_gather(gk_v, [flat])
                dx = nx - ox
                dy = ny - oy
                dz = nz - oz
                d2 = dx * dx + dy * dy + dz * dz + EPS
                rinv = _rsqrt(d2)
                dist = d2 * rinv
                st = dist - olj
                kl = jnp.exp(LN10 * kj)
                aa = jnp.abs(st) + EPS
                sq = aa * _rsqrt(aa)
                fm = kl * jnp.sign(st) * sq
                coef = fm * rinv
                ax = ax + coef * dx
                ay = ay + coef * dy
                az = az + coef * dz

            invm = jnp.exp(-LN10 * mlg)
            vnx = vx + (ax * invm) * DT
            vny = vy + (ay * invm + GRAV_Y) * DT
            vnz = vz + (az * invm) * DT
            xnx = ox + vnx * DT
            xny = oy + vny * DT
            xnz = oz + vnz * DT
            below = xny < GROUND
            xny = jnp.where(below, GROUND, xny)
            vny = jnp.where(below, -vny * REBOUND, vny)

            r6 = rr * 6
            plsc.store_scatter(out_v, [r6], xnx)
            plsc.store_scatter(out_v, [r6 + 1], xny)
            plsc.store_scatter(out_v, [r6 + 2], xnz)
            plsc.store_scatter(out_v, [r6 + 3], vnx)
            plsc.store_scatter(out_v, [r6 + 4], vny)
            plsc.store_scatter(out_v, [r6 + 5], vnz)
            return u

        lax.fori_loop(0, CG, group_body, 0)
        pltpu.make_async_copy(
            out_v, out.at[pl.ds(nb * 6, CN * 6)], osem).start()

    gather_round(0)

    def round_body(r, carry):
        plsc.subcore_barrier()
        own_t = (r * NS + sid) * NC + cid
        own_ok = own_t < NSLOT

        @pl.when(own_ok)
        def _():
            fire_compute_loads(r)

        @pl.when(r + 1 < ROUNDS)
        def _():
            gather_round(r + 1)

        @pl.when(own_ok)
        def _():
            compute_round(r)

        return carry

    lax.fori_loop(0, ROUNDS, round_body, 0)
    pltpu.make_async_copy(
        out_v, out.at[pl.ds(0, CN * 6)], osem).wait()


def kernel(xyz, velocity, origin_len, global_k, global_m, knn_index):
    xs = xyz[:, 0]
    ys = xyz[:, 1]
    zs = xyz[:, 2]
    xyzf = xyz.reshape(N * 3)
    velf = velocity.reshape(N * 3)
    olf = origin_len.reshape(N * K)
    gkf = global_k.reshape(N * K)
    knnf = knn_index.astype(jnp.int32).reshape(N * K)
    outf = _sc_step(xs, ys, zs, xyzf, velf, olf, gkf,
                    global_m.astype(jnp.float32), knnf)
    return outf.reshape(N, 6)


# speedup vs baseline: 2.1020x; 2.1020x over previous
import functools

import jax
import jax.numpy as jnp
import numpy as np
from jax import lax
from jax.experimental import pallas as pl
from jax.experimental.pallas import tpu as pltpu
from jax.experimental.pallas import tpu_sc as plsc

N = 100000
K = 16
L = 16
NC = 2
NS = 16
NW = NC * NS
CG = 24
CN = CG * L
NSLOT = -(-N // CN)
TAIL_NB = N - CN
NFULL = NSLOT - (NSLOT // NW) * NW
CNT_MAX = NSLOT // NW + 1
NIDX = CN * K // 128

DT = np.float32(0.01)
EPS = np.float32(1e-14)
GROUND = np.float32(-2.0)
REBOUND = np.float32(0.1)
GRAV_Y = np.float32(-9.8)
LN10 = np.float32(2.302585092994046)


def _rsqrt(x):
    i = lax.bitcast_convert_type(x, jnp.int32)
    i = np.int32(0x5F3759DF) - lax.shift_right_logical(i, 1)
    y = lax.bitcast_convert_type(i, jnp.float32)
    for _ in range(3):
        y = y * (np.float32(1.5) - np.float32(0.5) * x * y * y)
    return y


_mesh = plsc.VectorSubcoreMesh(core_axis_name="c", subcore_axis_name="s")

_BUF = lambda shape, dt: [pltpu.VMEM(shape, dt) for _ in range(2)]

@functools.partial(
    pl.kernel,
    out_type=jax.ShapeDtypeStruct((N * 6,), jnp.float32),
    mesh=_mesh,
    compiler_params=pltpu.CompilerParams(needs_layout_passes=False),
    scratch_types=[
        _BUF((CN * K,), jnp.int32),
        _BUF((CN * K,), jnp.float32),
        _BUF((CN * K,), jnp.float32),
        _BUF((CN * K,), jnp.float32),
        _BUF((CN,), jnp.float32),
        _BUF((CN,), jnp.float32),
        _BUF((CN,), jnp.float32),
        _BUF((CN * 3,), jnp.float32),
        _BUF((CN * K,), jnp.float32),
        _BUF((CN * K,), jnp.float32),
        _BUF((CN,), jnp.float32),
        _BUF((CN * 6,), jnp.float32),
        [pltpu.SemaphoreType.DMA for _ in range(2)],
        [pltpu.SemaphoreType.DMA for _ in range(2)],
        [pltpu.SemaphoreType.DMA for _ in range(2)],
        [pltpu.SemaphoreType.DMA for _ in range(2)],
        pltpu.VMEM_SHARED((N,), jnp.float32),
        pltpu.VMEM_SHARED((N,), jnp.float32),
        pltpu.VMEM_SHARED((N,), jnp.float32),
        pltpu.VMEM(((N // NS) // 8 * 8,), jnp.float32),
    ],
)
def _sc_step(xs, ys, zs, velf, olf, gkf, gm, knnf, out,
             knn_v, rx_v, ry_v, rz_v, ox_v, oy_v, oz_v,
             vel_v, ol_v, gk_v, gm_v, out_v,
             gsem, ksem, lsem, osem, sx, sy, sz, bounce):
    cid = lax.axis_index("c")
    sid = lax.axis_index("s")
    wid = sid * NC + cid
    cnt = jnp.where(wid < NFULL, CNT_MAX, CNT_MAX - 1)

    ii = lax.iota(jnp.int32, L)

    SL = (N // NS) // 8 * 8
    soff = sid * SL
    soff = pl.multiple_of(soff, 8)
    for src_hbm, dst_sp in ((xs, sx), (ys, sy), (zs, sz)):
        pltpu.sync_copy(src_hbm.at[pl.ds(soff, SL)], bounce)
        pltpu.sync_copy(bounce, dst_sp.at[pl.ds(soff, SL)])

    @pl.when(sid == NS - 1)
    def _():
        rem = N - SL * NS
        for src_hbm, dst_sp in ((xs, sx), (ys, sy), (zs, sz)):
            pltpu.sync_copy(src_hbm.at[pl.ds(SL * NS, rem)],
                            bounce.at[pl.ds(0, rem)])
            pltpu.sync_copy(bounce.at[pl.ds(0, rem)],
                            dst_sp.at[pl.ds(SL * NS, rem)])

    plsc.subcore_barrier()

    def node_base(m):
        t = wid + m * NW
        nb = jnp.where(t == NSLOT - 1, TAIL_NB, t * CN)
        return pl.multiple_of(nb, 32)

    def lin_copies(nb, b):
        return [
            pltpu.make_async_copy(xs.at[pl.ds(nb, CN)], ox_v[b], lsem[b]),
            pltpu.make_async_copy(ys.at[pl.ds(nb, CN)], oy_v[b], lsem[b]),
            pltpu.make_async_copy(zs.at[pl.ds(nb, CN)], oz_v[b], lsem[b]),
            pltpu.make_async_copy(
                velf.at[pl.ds(nb * 3, CN * 3)], vel_v[b], lsem[b]),
            pltpu.make_async_copy(
                olf.at[pl.ds(nb * K, CN * K)], ol_v[b], lsem[b]),
            pltpu.make_async_copy(
                gkf.at[pl.ds(nb * K, CN * K)], gk_v[b], lsem[b]),
            pltpu.make_async_copy(gm.at[pl.ds(nb, CN)], gm_v[b], lsem[b]),
        ]

    def issue(m, b):
        nb = node_base(m)
        kcp = pltpu.make_async_copy(
            knnf.at[pl.ds(nb * K, CN * K)], knn_v[b], ksem[b])
        kcp.start()
        for cp in lin_copies(nb, b):
            cp.start()
        kcp.wait()

        pltpu.make_async_copy(sx.at[knn_v[b]], rx_v[b], gsem[b]).start()
        pltpu.make_async_copy(sy.at[knn_v[b]], ry_v[b], gsem[b]).start()
        pltpu.make_async_copy(sz.at[knn_v[b]], rz_v[b], gsem[b]).start()

    def finish(m, b):
        nb = node_base(m)

        for r_v in (rx_v[b], ry_v[b], rz_v[b]):
            pltpu.make_async_copy(sx.at[knn_v[b]], r_v, gsem[b]).wait()
        for cp in lin_copies(nb, b):
            cp.wait()

        @pl.when(m >= 2)
        def _():
            pltpu.make_async_copy(
                out_v[b], out.at[pl.ds(nb * 6, CN * 6)], osem[b]).wait()

        def group_body(g, u):
            r = g * L + ii
            ox = plsc.load_gather(ox_v[b], [r])
            oy = plsc.load_gather(oy_v[b], [r])
            oz = plsc.load_gather(oz_v[b], [r])
            r3 = r * 3
            vx = plsc.load_gather(vel_v[b], [r3])
            vy = plsc.load_gather(vel_v[b], [r3 + 1])
            vz = plsc.load_gather(vel_v[b], [r3 + 2])
            mlg = plsc.load_gather(gm_v[b], [r])

            ax = jnp.zeros((L,), jnp.float32)
            ay = jnp.zeros((L,), jnp.float32)
            az = jnp.zeros((L,), jnp.float32)
            rk = r * K
            for j in range(K):
                flat = rk + j
                nx = plsc.load_gather(rx_v[b], [flat])
                ny = plsc.load_gather(ry_v[b], [flat])
                nz = plsc.load_gather(rz_v[b], [flat])
                olj = plsc.load_gather(ol_v[b], [flat])
                kj = plsc.load_gather(gk_v[b], [flat])
                dx = nx - ox
                dy = ny - oy
                dz = nz - oz
                d2 = dx * dx + dy * dy + dz * dz + EPS
                rinv = _rsqrt(d2)
                dist = d2 * rinv
                st = dist - olj
                kl = jnp.exp(LN10 * kj)
                a = jnp.abs(st) + EPS
                sq = a * _rsqrt(a)
                fm = kl * jnp.sign(st) * sq
                coef = fm * rinv
                ax = ax + coef * dx
                ay = ay + coef * dy
                az = az + coef * dz

            invm = jnp.exp(-LN10 * mlg)
            vnx = vx + (ax * invm) * DT
            vny = vy + (ay * invm + GRAV_Y) * DT
            vnz = vz + (az * invm) * DT
            xnx = ox + vnx * DT
            xny = oy + vny * DT
            xnz = oz + vnz * DT
            below = xny < GROUND
            xny = jnp.where(below, GROUND, xny)
            vny = jnp.where(below, -vny * REBOUND, vny)

            r6 = r * 6
            plsc.store_scatter(out_v[b], [r6], xnx)
            plsc.store_scatter(out_v[b], [r6 + 1], xny)
            plsc.store_scatter(out_v[b], [r6 + 2], xnz)
            plsc.store_scatter(out_v[b], [r6 + 3], vnx)
            plsc.store_scatter(out_v[b], [r6 + 4], vny)
            plsc.store_scatter(out_v[b], [r6 + 5], vnz)
            return u

        lax.fori_loop(0, CG, group_body, 0)
        pltpu.make_async_copy(
            out_v[b], out.at[pl.ds(nb * 6, CN * 6)], osem[b]).start()

    issue(0, 0)

    def pair_body(m0, carry):
        for b in (0, 1):
            m = m0 * 2 + b

            @pl.when(m + 1 < cnt)
            def _():
                issue(m + 1, 1 - b)

            @pl.when(m < cnt)
            def _():
                finish(m, b)
        return carry

    lax.fori_loop(0, CNT_MAX // 2 + 1, pair_body, 0)

    for b in (0, 1):
        pltpu.make_async_copy(
            out_v[b], out.at[pl.ds(0, CN * 6)], osem[b]).wait()


def kernel(xyz, velocity, origin_len, global_k, global_m, knn_index):
    xs = xyz[:, 0]
    ys = xyz[:, 1]
    zs = xyz[:, 2]
    velf = velocity.reshape(N * 3)
    olf = origin_len.reshape(N * K)
    gkf = global_k.reshape(N * K)
    knnf = knn_index.astype(jnp.int32).reshape(N * K)
    outf = _sc_step(xs, ys, zs, velf, olf, gkf,
                    global_m.astype(jnp.float32), knnf)
    return outf.reshape(N, 6)


# speedup vs baseline: 2.1315x; 1.0141x over previous
import functools

import jax
import jax.numpy as jnp
import numpy as np
from jax import lax
from jax.experimental import pallas as pl
from jax.experimental.pallas import tpu as pltpu
from jax.experimental.pallas import tpu_sc as plsc

N = 100000
K = 16
L = 16
NC = 2
NS = 16
NW = NC * NS
CG = 16
CN = CG * L
NSLOT = -(-N // CN)
TAIL_NB = N - CN
NFULL = NSLOT - (NSLOT // NW) * NW
CNT_MAX = NSLOT // NW + 1
NIDX = CN * K // 128

DT = np.float32(0.01)
EPS = np.float32(1e-14)
GROUND = np.float32(-2.0)
REBOUND = np.float32(0.1)
GRAV_Y = np.float32(-9.8)
LN10 = np.float32(2.302585092994046)


def _rsqrt(x):
    i = lax.bitcast_convert_type(x, jnp.int32)
    i = np.int32(0x5F3759DF) - lax.shift_right_logical(i, 1)
    y = lax.bitcast_convert_type(i, jnp.float32)
    for _ in range(3):
        y = y * (np.float32(1.5) - np.float32(0.5) * x * y * y)
    return y


_mesh = plsc.VectorSubcoreMesh(core_axis_name="c", subcore_axis_name="s")

_BUF = lambda shape, dt: [pltpu.VMEM(shape, dt) for _ in range(2)]

@functools.partial(
    pl.kernel,
    out_type=jax.ShapeDtypeStruct((N * 6,), jnp.float32),
    mesh=_mesh,
    compiler_params=pltpu.CompilerParams(needs_layout_passes=False),
    scratch_types=[
        _BUF((CN * K,), jnp.int32),
        _BUF((CN * K,), jnp.float32),
        _BUF((CN * K,), jnp.float32),
        _BUF((CN * K,), jnp.float32),
        _BUF((CN,), jnp.float32),
        _BUF((CN,), jnp.float32),
        _BUF((CN,), jnp.float32),
        _BUF((CN * 3,), jnp.float32),
        _BUF((CN * K,), jnp.float32),
        _BUF((CN * K,), jnp.float32),
        _BUF((CN,), jnp.float32),
        _BUF((CN * 6,), jnp.float32),
        [pltpu.SemaphoreType.DMA for _ in range(2)],
        [pltpu.SemaphoreType.DMA for _ in range(2)],
        [pltpu.SemaphoreType.DMA for _ in range(2)],
        [pltpu.SemaphoreType.DMA for _ in range(2)],
        pltpu.VMEM_SHARED((N,), jnp.float32),
        pltpu.VMEM_SHARED((N,), jnp.float32),
        pltpu.VMEM_SHARED((N,), jnp.float32),
        pltpu.VMEM(((N // NS) // 8 * 8,), jnp.float32),
    ],
)
def _sc_step(xs, ys, zs, velf, olf, gkf, gm, knnf, out,
             knn_v, rx_v, ry_v, rz_v, ox_v, oy_v, oz_v,
             vel_v, ol_v, gk_v, gm_v, out_v,
             gsem, ksem, lsem, osem, sx, sy, sz, bounce):
    cid = lax.axis_index("c")
    sid = lax.axis_index("s")
    wid = sid * NC + cid
    cnt = jnp.where(wid < NFULL, CNT_MAX, CNT_MAX - 1)

    ii = lax.iota(jnp.int32, L)

    SL = (N // NS) // 8 * 8
    soff = sid * SL
    soff = pl.multiple_of(soff, 8)
    for src_hbm, dst_sp in ((xs, sx), (ys, sy), (zs, sz)):
        pltpu.sync_copy(src_hbm.at[pl.ds(soff, SL)], bounce)
        pltpu.sync_copy(bounce, dst_sp.at[pl.ds(soff, SL)])

    @pl.when(sid == NS - 1)
    def _():
        rem = N - SL * NS
        for src_hbm, dst_sp in ((xs, sx), (ys, sy), (zs, sz)):
            pltpu.sync_copy(src_hbm.at[pl.ds(SL * NS, rem)],
                            bounce.at[pl.ds(0, rem)])
            pltpu.sync_copy(bounce.at[pl.ds(0, rem)],
                            dst_sp.at[pl.ds(SL * NS, rem)])

    plsc.subcore_barrier()

    def node_base(m):
        t = wid + m * NW
        nb = jnp.where(t == NSLOT - 1, TAIL_NB, t * CN)
        return pl.multiple_of(nb, 32)

    def lin_copies(nb, b):
        return [
            pltpu.make_async_copy(xs.at[pl.ds(nb, CN)], ox_v[b], lsem[b]),
            pltpu.make_async_copy(ys.at[pl.ds(nb, CN)], oy_v[b], lsem[b]),
            pltpu.make_async_copy(zs.at[pl.ds(nb, CN)], oz_v[b], lsem[b]),
            pltpu.make_async_copy(
                velf.at[pl.ds(nb * 3, CN * 3)], vel_v[b], lsem[b]),
            pltpu.make_async_copy(
                olf.at[pl.ds(nb * K, CN * K)], ol_v[b], lsem[b]),
            pltpu.make_async_copy(
                gkf.at[pl.ds(nb * K, CN * K)], gk_v[b], lsem[b]),
            pltpu.make_async_copy(gm.at[pl.ds(nb, CN)], gm_v[b], lsem[b]),
        ]

    def issue(m, b):
        nb = node_base(m)
        kcp = pltpu.make_async_copy(
            knnf.at[pl.ds(nb * K, CN * K)], knn_v[b], ksem[b])
        kcp.start()
        for cp in lin_copies(nb, b):
            cp.start()
        kcp.wait()

        pltpu.make_async_copy(sx.at[knn_v[b]], rx_v[b], gsem[b]).start()
        pltpu.make_async_copy(sy.at[knn_v[b]], ry_v[b], gsem[b]).start()
        pltpu.make_async_copy(sz.at[knn_v[b]], rz_v[b], gsem[b]).start()

    def finish(m, b):
        nb = node_base(m)

        for r_v in (rx_v[b], ry_v[b], rz_v[b]):
            pltpu.make_async_copy(sx.at[knn_v[b]], r_v, gsem[b]).wait()
        for cp in lin_copies(nb, b):
            cp.wait()

        @pl.when(m >= 2)
        def _():
            pltpu.make_async_copy(
                out_v[b], out.at[pl.ds(nb * 6, CN * 6)], osem[b]).wait()

        def group_body(g, u):
            r = g * L + ii
            ox = plsc.load_gather(ox_v[b], [r])
            oy = plsc.load_gather(oy_v[b], [r])
            oz = plsc.load_gather(oz_v[b], [r])
            r3 = r * 3
            vx = plsc.load_gather(vel_v[b], [r3])
            vy = plsc.load_gather(vel_v[b], [r3 + 1])
            vz = plsc.load_gather(vel_v[b], [r3 + 2])
            mlg = plsc.load_gather(gm_v[b], [r])

            ax = jnp.zeros((L,), jnp.float32)
            ay = jnp.zeros((L,), jnp.float32)
            az = jnp.zeros((L,), jnp.float32)
            rk = r * K
            for j in range(K):
                flat = rk + j
                nx = plsc.load_gather(rx_v[b], [flat])
                ny = plsc.load_gather(ry_v[b], [flat])
                nz = plsc.load_gather(rz_v[b], [flat])
                olj = plsc.load_gather(ol_v[b], [flat])
                kj = plsc.load_gather(gk_v[b], [flat])
                dx = nx - ox
                dy = ny - oy
                dz = nz - oz
                d2 = dx * dx + dy * dy + dz * dz + EPS
                rinv = _rsqrt(d2)
                dist = d2 * rinv
                st = dist - olj
                kl = jnp.exp(LN10 * kj)
                a = jnp.abs(st) + EPS
                sq = a * _rsqrt(a)
                fm = kl * jnp.sign(st) * sq
                coef = fm * rinv
                ax = ax + coef * dx
                ay = ay + coef * dy
                az = az + coef * dz

            invm = jnp.exp(-LN10 * mlg)
            vnx = vx + (ax * invm) * DT
            vny = vy + (ay * invm + GRAV_Y) * DT
            vnz = vz + (az * invm) * DT
            xnx = ox + vnx * DT
            xny = oy + vny * DT
            xnz = oz + vnz * DT
            below = xny < GROUND
            xny = jnp.where(below, GROUND, xny)
            vny = jnp.where(below, -vny * REBOUND, vny)

            r6 = r * 6
            plsc.store_scatter(out_v[b], [r6], xnx)
            plsc.store_scatter(out_v[b], [r6 + 1], xny)
            plsc.store_scatter(out_v[b], [r6 + 2], xnz)
            plsc.store_scatter(out_v[b], [r6 + 3], vnx)
            plsc.store_scatter(out_v[b], [r6 + 4], vny)
            plsc.store_scatter(out_v[b], [r6 + 5], vnz)
            return u

        lax.fori_loop(0, CG, group_body, 0)
        pltpu.make_async_copy(
            out_v[b], out.at[pl.ds(nb * 6, CN * 6)], osem[b]).start()

    issue(0, 0)

    def pair_body(m0, carry):
        for b in (0, 1):
            m = m0 * 2 + b

            @pl.when(m + 1 < cnt)
            def _():
                issue(m + 1, 1 - b)

            @pl.when(m < cnt)
            def _():
                finish(m, b)
        return carry

    lax.fori_loop(0, CNT_MAX // 2 + 1, pair_body, 0)

    for b in (0, 1):
        pltpu.make_async_copy(
            out_v[b], out.at[pl.ds(0, CN * 6)], osem[b]).wait()


def kernel(xyz, velocity, origin_len, global_k, global_m, knn_index):
    xs = xyz[:, 0]
    ys = xyz[:, 1]
    zs = xyz[:, 2]
    velf = velocity.reshape(N * 3)
    olf = origin_len.reshape(N * K)
    gkf = global_k.reshape(N * K)
    knnf = knn_index.astype(jnp.int32).reshape(N * K)
    outf = _sc_step(xs, ys, zs, velf, olf, gkf,
                    global_m.astype(jnp.float32), knnf)
    return outf.reshape(N, 6)
